# baseline (speedup 1.0000x reference)
import functools
import jax
import jax.numpy as jnp
from jax import lax
from jax.experimental import pallas as pl
from jax.experimental.pallas import tpu as pltpu
from jax.experimental.pallas import tpu_sc as plsc

D = 2048
NTOK = 4 * 4096
T = 16
ROWS_PER_W = NTOK // 32
NTILES = ROWS_PER_W // T


def _sc_body(w_hbm, x_hbm, o_hbm, w_v, x_v, o_v):
    nc = 2
    wid = lax.axis_index("s") * nc + lax.axis_index("c")
    pltpu.sync_copy(w_hbm, w_v)
    u2 = w_v[pl.ds(0, 16)]
    u4 = w_v[pl.ds(16, 16)]
    u8 = w_v[pl.ds(32, 16)]
    d2 = w_v[pl.ds(48, 16)]
    d4 = w_v[pl.ds(64, 16)]
    d8 = w_v[pl.ds(80, 16)]
    lane = lax.iota(jnp.int32, 16)

    def up_acc(base, j, acc):
        for lg, uw in ((1, u2), (2, u4), (3, u8)):
            s = 1 << lg
            m = (j & (s - 1)) == 0
            g = plsc.load_gather(x_v, [base + (j >> lg)])
            acc = acc + uw * jnp.where(m, g, 0.0)
        return acc

    def down_acc(base, j, acc, strides, mask_first):
        for lg, dw in strides:
            s = 1 << lg
            p = jnp.zeros((16,), jnp.float32)
            for i in range(s):
                p = p + plsc.load_gather(x_v, [base + (j << lg) + i])
            if mask_first:
                p = jnp.where(j >= 1, p, 0.0)
            acc = acc + dw * p
        return acc

    def do_tile(t, _):
        row0 = wid * ROWS_PER_W + t * T
        pltpu.sync_copy(x_hbm.at[pl.ds(row0 * D, T * D)], x_v)

        def do_row(r, _):
            base = r * D

            def chunk(c, strides, mask_first):
                j = c * 16 + lane
                acc = x_v[pl.ds(base + c * 16, 16)]
                acc = up_acc(base, j, acc)
                acc = down_acc(base, j, acc, strides, mask_first)
                o_v[pl.ds(base + c * 16, 16)] = acc

            chunk(0, ((1, d2), (2, d4), (3, d8)), True)
            lax.fori_loop(1, 16, lambda c, _: (chunk(c, ((1, d2), (2, d4), (3, d8)), False), 0)[1], 0)
            lax.fori_loop(16, 32, lambda c, _: (chunk(c, ((1, d2), (2, d4)), False), 0)[1], 0)
            lax.fori_loop(32, 64, lambda c, _: (chunk(c, ((1, d2),), False), 0)[1], 0)
            lax.fori_loop(64, 128, lambda c, _: (chunk(c, (), False), 0)[1], 0)
            return 0

        lax.fori_loop(0, T, do_row, 0)
        pltpu.sync_copy(o_v, o_hbm.at[pl.ds(row0 * D, T * D)])
        return 0

    lax.fori_loop(0, NTILES, do_tile, 0)


def kernel(x, up_weights, down_weights):
    B, S, d = x.shape
    xf = x.reshape(B * S * d)
    w = jnp.concatenate([jax.nn.sigmoid(up_weights), jax.nn.sigmoid(down_weights)])
    wb = jnp.broadcast_to(w[:, None], (6, 16)).reshape(96)
    mesh = plsc.VectorSubcoreMesh(core_axis_name="c", subcore_axis_name="s")
    run = functools.partial(
        pl.kernel,
        mesh=mesh,
        compiler_params=pltpu.CompilerParams(needs_layout_passes=False),
        out_type=jax.ShapeDtypeStruct((B * S * d,), jnp.float32),
        scratch_types=[
            pltpu.VMEM((96,), jnp.float32),
            pltpu.VMEM((T * D,), jnp.float32),
            pltpu.VMEM((T * D,), jnp.float32),
        ],
    )(_sc_body)
    out = run(wb, xf)
    return out.reshape(B, S, d)
